# baseline (speedup 1.0000x reference)
import functools

import jax
import jax.numpy as jnp
from jax import lax
from jax.experimental import pallas as pl
from jax.experimental.pallas import tpu as pltpu
from jax.experimental.pallas import tpu_sc as plsc

B, H, W = 4, 512, 512
HW = H * W
NC, NS, L = 2, 16, 16
NW = NC * NS
NPAIR = 8192
PAIRS_W = 1024
NVAL = 2 * PAIRS_W
CHUNK = 128
NCHUNK = NVAL // CHUNK
BLK = 256
NBLK = PAIRS_W // BLK


def _sc_gather(g0_hbm, g1_hbm, x_hbm, out_hbm, idx_v, vals_v, vi_v, sem, osem):
    wid = lax.axis_index("s") * NC + lax.axis_index("c")
    half = wid // 16
    w2 = wid % 16
    b = w2 // 4
    q = w2 % 4

    copies = []
    for h in range(2):
        hs = pl.ds(h * (NVAL // 2), NVAL // 2)

        @pl.when(half == 0)
        def _(hs=hs):
            pltpu.sync_copy(g0_hbm.at[b, q, hs], idx_v.at[hs])

        @pl.when(half == 1)
        def _(hs=hs):
            pltpu.sync_copy(g1_hbm.at[b, q, hs], idx_v.at[hs])

        for j in range(h * NCHUNK // 2, (h + 1) * NCHUNK // 2):
            copies.append(
                pltpu.async_copy(
                    x_hbm.at[idx_v.at[pl.ds(j * CHUNK, CHUNK)]],
                    vals_v.at[pl.ds(j * CHUNK, CHUNK)],
                    sem,
                )
            )
    for c in copies:
        c.wait()

    lane = lax.iota(jnp.int32, L)
    rows0 = lax.shift_right_logical(lane, 1)
    cols = lax.bitwise_and(lane, 1)
    pair0 = half * (NPAIR // 2) + q * PAIRS_W
    out_blk = lambda blk: out_hbm.at[b, pl.ds(pair0 + blk * BLK, BLK), :]

    def _block(blk, carry):
        slot = lax.rem(blk, 2)
        slot_v = lax.broadcast(slot, (L,))

        @pl.when(blk >= 2)
        def _():
            pltpu.make_async_copy(out_blk(blk), vi_v.at[slot], osem).wait()

        for i in range(2 * BLK // L):
            v = vals_v[pl.ds(blk * 2 * BLK + i * L, L)]
            plsc.store_scatter(
                vi_v, [slot_v, rows0 + i * (L // 2), cols], v
            )
        pltpu.async_copy(vi_v.at[slot], out_blk(blk), osem)
        return carry

    lax.fori_loop(0, NBLK, _block, 0)
    pltpu.make_async_copy(out_blk(NBLK - 2), vi_v.at[0], osem).wait()
    pltpu.make_async_copy(out_blk(NBLK - 1), vi_v.at[1], osem).wait()


@jax.jit
def kernel(X, cof0, cof1):
    b = X.shape[0]
    xflat = X.reshape(-1)
    base = (jnp.arange(b, dtype=jnp.int32) * HW)[:, None, None]
    g0 = (cof0.astype(jnp.int32) + base).reshape(b, 4, NVAL)
    g1 = (cof1.astype(jnp.int32) + base).reshape(b, 4, NVAL)

    mesh = plsc.VectorSubcoreMesh(core_axis_name="c", subcore_axis_name="s")
    run = functools.partial(
        pl.kernel,
        mesh=mesh,
        compiler_params=pltpu.CompilerParams(
            needs_layout_passes=False, use_tc_tiling_on_sc=True
        ),
        out_type=jax.ShapeDtypeStruct((b, NPAIR, 2), jnp.float32),
        scratch_types=[
            pltpu.VMEM((NVAL,), jnp.int32),
            pltpu.VMEM((NVAL,), jnp.float32),
            pltpu.VMEM((2, BLK, 2), jnp.float32),
            pltpu.SemaphoreType.DMA,
            pltpu.SemaphoreType.DMA,
        ],
    )(_sc_gather)
    return run(g0, g1, xflat)


# speedup vs baseline: 1.0015x; 1.0015x over previous
import functools

import jax
import jax.numpy as jnp
from jax import lax
from jax.experimental import pallas as pl
from jax.experimental.pallas import tpu as pltpu
from jax.experimental.pallas import tpu_sc as plsc

B, H, W = 4, 512, 512
HW = H * W
NC, NS, L = 2, 16, 16
NW = NC * NS
NPAIR = 8192
PAIRS_W = 1024
NVAL = 2 * PAIRS_W
CHUNK = 128
NCHUNK = NVAL // CHUNK
BLK = 256
NBLK = PAIRS_W // BLK


def _sc_gather(g_hbm, x_hbm, out_hbm, idx_v, vals_v, vi_v, sem, osem):
    wid = lax.axis_index("s") * NC + lax.axis_index("c")
    half = wid // 16
    w2 = wid % 16
    b = w2 // 4
    q = w2 % 4

    row = half * 4 + q
    copies = []
    for h in range(2):
        hs = pl.ds(h * (NVAL // 2), NVAL // 2)
        pltpu.sync_copy(g_hbm.at[b, row, hs], idx_v.at[hs])
        for j in range(h * NCHUNK // 2, (h + 1) * NCHUNK // 2):
            copies.append(
                pltpu.async_copy(
                    x_hbm.at[idx_v.at[pl.ds(j * CHUNK, CHUNK)]],
                    vals_v.at[pl.ds(j * CHUNK, CHUNK)],
                    sem,
                )
            )
    for c in copies:
        c.wait()

    lane = lax.iota(jnp.int32, L)
    rows0 = lax.shift_right_logical(lane, 1)
    cols = lax.bitwise_and(lane, 1)
    pair0 = half * (NPAIR // 2) + q * PAIRS_W
    out_blk = lambda blk: out_hbm.at[b, pl.ds(pair0 + blk * BLK, BLK), :]

    def _block(blk, carry):
        slot = lax.rem(blk, 2)
        slot_v = lax.broadcast(slot, (L,))

        @pl.when(blk >= 2)
        def _():
            pltpu.make_async_copy(out_blk(blk), vi_v.at[slot], osem).wait()

        for i in range(2 * BLK // L):
            v = vals_v[pl.ds(blk * 2 * BLK + i * L, L)]
            plsc.store_scatter(
                vi_v, [slot_v, rows0 + i * (L // 2), cols], v
            )
        pltpu.async_copy(vi_v.at[slot], out_blk(blk), osem)
        return carry

    lax.fori_loop(0, NBLK, _block, 0)
    pltpu.make_async_copy(out_blk(NBLK - 2), vi_v.at[0], osem).wait()
    pltpu.make_async_copy(out_blk(NBLK - 1), vi_v.at[1], osem).wait()


@jax.jit
def kernel(X, cof0, cof1):
    b = X.shape[0]
    xflat = X.reshape(-1)
    base = (jnp.arange(b, dtype=jnp.int32) * HW)[:, None, None]
    g = jnp.concatenate(
        [
            (cof0.astype(jnp.int32) + base).reshape(b, 4, NVAL),
            (cof1.astype(jnp.int32) + base).reshape(b, 4, NVAL),
        ],
        axis=1,
    )

    mesh = plsc.VectorSubcoreMesh(core_axis_name="c", subcore_axis_name="s")
    run = functools.partial(
        pl.kernel,
        mesh=mesh,
        compiler_params=pltpu.CompilerParams(needs_layout_passes=False),
        out_type=jax.ShapeDtypeStruct((b, NPAIR, 2), jnp.float32),
        scratch_types=[
            pltpu.VMEM((NVAL,), jnp.int32),
            pltpu.VMEM((NVAL,), jnp.float32),
            pltpu.VMEM((2, BLK, 2), jnp.float32),
            pltpu.SemaphoreType.DMA,
            pltpu.SemaphoreType.DMA,
        ],
    )(_sc_gather)
    return run(g, xflat)


# speedup vs baseline: 1.0233x; 1.0218x over previous
import functools

import jax
import jax.numpy as jnp
from jax import lax
from jax.experimental import pallas as pl
from jax.experimental.pallas import tpu as pltpu
from jax.experimental.pallas import tpu_sc as plsc

B, H, W = 4, 512, 512
HW = H * W
NC, NS, L = 2, 16, 16
NW = NC * NS
NPAIR = 8192
PAIRS_W = 1024
NVAL = 2 * PAIRS_W
CHUNK = 128
NCHUNK = NVAL // CHUNK
BLK = 256
NBLK = PAIRS_W // BLK


def _sc_gather(g_hbm, x_hbm, out_hbm, idx_v, vals_v, vi_v, sem, sem2, osem):
    wid = lax.axis_index("s") * NC + lax.axis_index("c")
    half = wid // 16
    w2 = wid % 16
    b = w2 // 4
    q = w2 % 4

    row = half * 4 + q
    sems = (sem, sem2)
    for h in range(2):
        hs = pl.ds(h * (NVAL // 2), NVAL // 2)
        pltpu.sync_copy(g_hbm.at[b, row, hs], idx_v.at[hs])
        for j in range(h * NCHUNK // 2, (h + 1) * NCHUNK // 2):
            pltpu.async_copy(
                x_hbm.at[idx_v.at[pl.ds(j * CHUNK, CHUNK)]],
                vals_v.at[pl.ds(j * CHUNK, CHUNK)],
                sems[h],
            )

    lane = lax.iota(jnp.int32, L)
    rows0 = lax.shift_right_logical(lane, 1)
    cols = lax.bitwise_and(lane, 1)
    pair0 = half * (NPAIR // 2) + q * PAIRS_W
    out_blk = lambda blk: out_hbm.at[b, pl.ds(pair0 + blk * BLK, BLK), :]

    def _block(blk, carry):
        slot = lax.rem(blk, 2)
        slot_v = lax.broadcast(slot, (L,))

        @pl.when(blk >= 2)
        def _():
            pltpu.make_async_copy(out_blk(blk), vi_v.at[slot], osem).wait()

        for i in range(2 * BLK // L):
            v = vals_v[pl.ds(blk * 2 * BLK + i * L, L)]
            plsc.store_scatter(
                vi_v, [slot_v, rows0 + i * (L // 2), cols], v
            )
        pltpu.async_copy(vi_v.at[slot], out_blk(blk), osem)
        return carry

    for h in range(2):
        for j in range(NCHUNK // 2):
            pltpu.make_async_copy(
                x_hbm.at[pl.ds(0, CHUNK)],
                vals_v.at[pl.ds(j * CHUNK, CHUNK)],
                sems[h],
            ).wait()
        lax.fori_loop(h * NBLK // 2, (h + 1) * NBLK // 2, _block, 0)

    pltpu.make_async_copy(out_blk(NBLK - 2), vi_v.at[0], osem).wait()
    pltpu.make_async_copy(out_blk(NBLK - 1), vi_v.at[1], osem).wait()


@jax.jit
def kernel(X, cof0, cof1):
    b = X.shape[0]
    xflat = X.reshape(-1)
    base = (jnp.arange(b, dtype=jnp.int32) * HW)[:, None, None]
    g = jnp.concatenate(
        [
            (cof0.astype(jnp.int32) + base).reshape(b, 4, NVAL),
            (cof1.astype(jnp.int32) + base).reshape(b, 4, NVAL),
        ],
        axis=1,
    )

    mesh = plsc.VectorSubcoreMesh(core_axis_name="c", subcore_axis_name="s")
    run = functools.partial(
        pl.kernel,
        mesh=mesh,
        compiler_params=pltpu.CompilerParams(needs_layout_passes=False),
        out_type=jax.ShapeDtypeStruct((b, NPAIR, 2), jnp.float32),
        scratch_types=[
            pltpu.VMEM((NVAL,), jnp.int32),
            pltpu.VMEM((NVAL,), jnp.float32),
            pltpu.VMEM((2, BLK, 2), jnp.float32),
            pltpu.SemaphoreType.DMA,
            pltpu.SemaphoreType.DMA,
            pltpu.SemaphoreType.DMA,
        ],
    )(_sc_gather)
    return run(g, xflat)


# speedup vs baseline: 1.0253x; 1.0020x over previous
import functools

import jax
import jax.numpy as jnp
from jax import lax
from jax.experimental import pallas as pl
from jax.experimental.pallas import tpu as pltpu
from jax.experimental.pallas import tpu_sc as plsc

B, H, W = 4, 512, 512
HW = H * W
NC, NS, L = 2, 16, 16
NW = NC * NS
NPAIR = 8192
PAIRS_W = 1024
NVAL = 2 * PAIRS_W
CHUNK = 128
NCHUNK = NVAL // CHUNK
BLK = 128
NBLK = PAIRS_W // BLK


def _sc_gather(g_hbm, x_hbm, out_hbm, idx_v, vals_v, vi_v, sem, sem2, sem3, sem4, osem):
    wid = lax.axis_index("s") * NC + lax.axis_index("c")
    half = wid // 16
    w2 = wid % 16
    b = w2 // 4
    q = w2 % 4

    row = half * 4 + q
    sems = (sem, sem2, sem3, sem4)
    for h in range(2):
        hs = pl.ds(h * (NVAL // 2), NVAL // 2)
        pltpu.sync_copy(g_hbm.at[b, row, hs], idx_v.at[hs])
        for j in range(h * NCHUNK // 2, (h + 1) * NCHUNK // 2):
            pltpu.async_copy(
                x_hbm.at[idx_v.at[pl.ds(j * CHUNK, CHUNK)]],
                vals_v.at[pl.ds(j * CHUNK, CHUNK)],
                sems[j * 4 // NCHUNK],
            )

    lane = lax.iota(jnp.int32, L)
    rows0 = lax.shift_right_logical(lane, 1)
    cols = lax.bitwise_and(lane, 1)
    pair0 = half * (NPAIR // 2) + q * PAIRS_W
    out_blk = lambda blk: out_hbm.at[b, pl.ds(pair0 + blk * BLK, BLK), :]

    def _block(blk, carry):
        slot = lax.rem(blk, 2)
        slot_v = lax.broadcast(slot, (L,))

        @pl.when(blk >= 2)
        def _():
            pltpu.make_async_copy(out_blk(blk), vi_v.at[slot], osem).wait()

        for i in range(2 * BLK // L):
            v = vals_v[pl.ds(blk * 2 * BLK + i * L, L)]
            plsc.store_scatter(
                vi_v, [slot_v, rows0 + i * (L // 2), cols], v
            )
        pltpu.async_copy(vi_v.at[slot], out_blk(blk), osem)
        return carry

    for grp in range(4):
        for j in range(NCHUNK // 4):
            pltpu.make_async_copy(
                x_hbm.at[pl.ds(0, CHUNK)],
                vals_v.at[pl.ds(j * CHUNK, CHUNK)],
                sems[grp],
            ).wait()
        lax.fori_loop(grp * NBLK // 4, (grp + 1) * NBLK // 4, _block, 0)

    pltpu.make_async_copy(out_blk(NBLK - 2), vi_v.at[0], osem).wait()
    pltpu.make_async_copy(out_blk(NBLK - 1), vi_v.at[1], osem).wait()


@jax.jit
def kernel(X, cof0, cof1):
    b = X.shape[0]
    xflat = X.reshape(-1)
    base = (jnp.arange(b, dtype=jnp.int32) * HW)[:, None, None]
    g = jnp.concatenate(
        [
            (cof0.astype(jnp.int32) + base).reshape(b, 4, NVAL),
            (cof1.astype(jnp.int32) + base).reshape(b, 4, NVAL),
        ],
        axis=1,
    )

    mesh = plsc.VectorSubcoreMesh(core_axis_name="c", subcore_axis_name="s")
    run = functools.partial(
        pl.kernel,
        mesh=mesh,
        compiler_params=pltpu.CompilerParams(needs_layout_passes=False),
        out_type=jax.ShapeDtypeStruct((b, NPAIR, 2), jnp.float32),
        scratch_types=[
            pltpu.VMEM((NVAL,), jnp.int32),
            pltpu.VMEM((NVAL,), jnp.float32),
            pltpu.VMEM((2, BLK, 2), jnp.float32),
            pltpu.SemaphoreType.DMA,
            pltpu.SemaphoreType.DMA,
            pltpu.SemaphoreType.DMA,
            pltpu.SemaphoreType.DMA,
            pltpu.SemaphoreType.DMA,
        ],
    )(_sc_gather)
    return run(g, xflat)
